# baseline (speedup 1.0000x reference)
import functools

import jax
import jax.numpy as jnp
from jax import lax
from jax.experimental import pallas as pl
from jax.experimental.pallas import tpu as pltpu
from jax.experimental.pallas import tpu_sc as plsc

N_NODES = 100000
N_EDGES = 3200000
EDGE_DIM = 16
HIDDEN = 8
TOPK = 16

NW = 32
SEG_PER_TILE = 3136
SEG_PAD = NW * SEG_PER_TILE

ROWS = N_EDGES // 8
R_BLK = 8000

C_A = 8192
EPT = N_EDGES // NW
C_B = 4000
GW = 80

_I32MAX = 2147483647


def _mlp_body(x_ref, w1_ref, b1_ref, w2_ref, b2_ref, p_ref):
    x = x_ref[...]
    h = jax.lax.dot(x, w1_ref[...], preferred_element_type=jnp.float32)
    h = jnp.maximum(h + b1_ref[...], 0.0)
    s = jax.lax.dot(h, w2_ref[...], preferred_element_type=jnp.float32)
    p_ref[...] = jnp.exp(s + b2_ref[...])


def _stage1(x, w1b, b1t, w2b, b2t):
    return pl.pallas_call(
        _mlp_body,
        grid=(ROWS // R_BLK,),
        in_specs=[
            pl.BlockSpec((R_BLK, 128), lambda i: (i, 0)),
            pl.BlockSpec((128, 64), lambda i: (0, 0)),
            pl.BlockSpec((1, 64), lambda i: (0, 0)),
            pl.BlockSpec((64, 8), lambda i: (0, 0)),
            pl.BlockSpec((1, 8), lambda i: (0, 0)),
        ],
        out_specs=pl.BlockSpec((R_BLK, 8), lambda i: (i, 0)),
        out_shape=jax.ShapeDtypeStruct((ROWS, 8), jnp.float32),
    )(x, w1b, b1t, w2b, b2t)


def _seg_stats_body(p_hbm, ids_hbm, pair_hbm,
                    pbuf, ibuf, probe, ptab):
    wid = lax.axis_index("s") * 2 + lax.axis_index("c")
    s0 = wid * SEG_PER_TILE
    lane = lax.iota(jnp.int32, 16)
    zeros16 = jnp.zeros((16,), jnp.float32)

    def lower_bound(target):
        def body(_, st):
            l, h = st
            upd = h - l > 1
            mid = jnp.clip((l + h) // 2, 0, N_EDGES - 1)
            mid16 = pl.multiple_of((mid // 16) * 16, 16)
            pltpu.sync_copy(ids_hbm.at[pl.ds(mid16, 16)], probe)
            v16 = probe[...]
            v = jnp.sum(jnp.where(lane == mid - mid16, v16, 0))
            lt = v < target
            return (jnp.where(upd & lt, mid, l),
                    jnp.where(upd & (~lt), mid, h))

        _, h = lax.fori_loop(0, 24, body,
                             (jnp.int32(-1), jnp.int32(N_EDGES)))
        return h

    lo = lower_bound(s0)
    hi = lower_bound(s0 + SEG_PER_TILE)

    def zinit(i, _):
        ptab[pl.ds(i * 16, 16)] = zeros16
        return 0

    lax.fori_loop(0, 2 * SEG_PER_TILE // 16, zinit, 0)

    def close_seg(scur, sacc, top):
        idx = 2 * (scur - s0) + lane
        m01 = (lane < 2) & (scur >= s0)
        thr16 = jnp.full((16,), jnp.min(top), jnp.float32)
        rv = 1.0 / jnp.full((16,), jnp.sum(sacc), jnp.float32)
        vals = jnp.where(lane == 0, thr16, rv)
        plsc.store_scatter(ptab, [idx], vals, mask=m01)

    cbase0 = (lo // 16) * 16

    def chunk_step(k, st):
        cbase = pl.multiple_of(cbase0 + k * C_A, 16)

        def active(st):
            scur, sacc, top = st
            start = pl.multiple_of(
                jnp.minimum(cbase, jnp.int32(N_EDGES - C_A)), 16)
            off0 = cbase - start
            pltpu.sync_copy(p_hbm.at[pl.ds(start, C_A)],
                            pbuf.at[pl.ds(0, C_A)])
            pltpu.sync_copy(ids_hbm.at[pl.ds(start, C_A)],
                            ibuf.at[pl.ds(0, C_A)])

            def win_body(j, wst):
                scur, sacc, top = wst
                woff = off0 + j * 16
                ids_v = ibuf[pl.ds(woff, 16)]
                p_v = pbuf[pl.ds(woff, 16)]
                gidx = cbase + j * 16 + lane
                valid = (gidx >= lo) & (gidx < hi)

                def acc(scur, sacc, top):
                    pm = jnp.where(valid & (ids_v == scur), p_v, 0.0)
                    snew = lax.sort(pm)
                    top2 = lax.sort(jnp.maximum(top, lax.rev(snew, (0,))))
                    return sacc + pm, top2

                sacc, top = acc(scur, sacc, top)

                def seg_cond(st2):
                    return jnp.any(valid & (ids_v > st2[0]))

                def seg_body(st2):
                    scur, sacc, top = st2
                    close_seg(scur, sacc, top)
                    scur = jnp.min(jnp.where(valid & (ids_v > scur), ids_v,
                                             _I32MAX))
                    sacc, top = acc(scur, zeros16, zeros16)
                    return scur, sacc, top

                return lax.while_loop(seg_cond, seg_body, (scur, sacc, top))

            return lax.fori_loop(0, C_A // 16, win_body, (scur, sacc, top))

        return lax.cond(cbase < hi, active, lambda s: s, st)

    max_chunks = (N_EDGES + C_A - 1) // C_A + 1
    scur, sacc, top = lax.fori_loop(
        0, max_chunks, chunk_step,
        (jnp.int32(-1), zeros16, zeros16))
    close_seg(scur, sacc, top)

    s0a = pl.multiple_of(2 * s0, 64)
    pltpu.sync_copy(ptab, pair_hbm.at[pl.ds(s0a, 2 * SEG_PER_TILE)])


def _stage2(p_flat, segment_ids):
    f = pl.kernel(
        _seg_stats_body,
        out_type=jax.ShapeDtypeStruct((2 * SEG_PAD,), jnp.float32),
        mesh=plsc.VectorSubcoreMesh(core_axis_name="c", subcore_axis_name="s"),
        scratch_types=[
            pltpu.VMEM((2 * C_A,), jnp.float32),
            pltpu.VMEM((2 * C_A,), jnp.int32),
            pltpu.VMEM((16,), jnp.int32),
            pltpu.VMEM((2 * SEG_PER_TILE,), jnp.float32),
        ],
        compiler_params=pltpu.CompilerParams(needs_layout_passes=False),
    )
    return f(p_flat, segment_ids)


BUF_B = 8192


def _apply_body(p_hbm, ids_hbm, thr_hbm, rden_hbm, out_hbm,
                pbuf, ibuf, tbuf, rbuf, tsl, rsl, obuf, sem, sem2):
    wid = lax.axis_index("s") * 2 + lax.axis_index("c")
    base = wid * EPT
    lane = lax.iota(jnp.int32, 16)

    def chunk(kc, _):
        cb = pl.multiple_of(base + kc * C_B, 16)
        cp_p = pltpu.async_copy(p_hbm.at[pl.ds(cb, C_B)], pbuf, sem2)
        pltpu.sync_copy(ids_hbm.at[pl.ds(cb, C_B)], ibuf)
        first = jnp.sum(jnp.where(lane == 0, ibuf[pl.ds(0, 16)], 0))
        last = jnp.sum(jnp.where(lane == 15, ibuf[pl.ds(C_B - 16, 16)], 0))
        base8 = pl.multiple_of((first // 8) * 8, 8)
        span = last - base8 + 1

        def narrow(_):
            cp_t = pltpu.async_copy(thr_hbm.at[pl.ds(base8, BUF_B)], tsl, sem)
            pltpu.sync_copy(rden_hbm.at[pl.ds(base8, BUF_B)], rsl)
            cp_t.wait()

            def win(j, _):
                sl = pl.ds(j * 16, 16)
                pv = pbuf[sl]
                lidx = ibuf[sl] - base8
                tv = plsc.load_gather(tsl, [lidx])
                rv = plsc.load_gather(rsl, [lidx])
                obuf[sl] = jnp.where(pv >= tv, pv * rv, 0.0)
                return 0

            lax.fori_loop(0, C_B // 16, win, 0)
            return 0

        def wide(_):
            ngrp = C_B // GW
            for g0 in range(0, ngrp, 8):
                cps = []
                for g in range(g0, min(g0 + 8, ngrp)):
                    sl = pl.ds(g * GW, GW)
                    cps.append(pltpu.async_copy(thr_hbm.at[ibuf.at[sl]],
                                                tbuf.at[sl], sem))
                    cps.append(pltpu.async_copy(rden_hbm.at[ibuf.at[sl]],
                                                rbuf.at[sl], sem))
                for cp in cps:
                    cp.wait()

            def win(j, _):
                sl = pl.ds(j * 16, 16)
                pv = pbuf[sl]
                obuf[sl] = jnp.where(pv >= tbuf[sl], pv * rbuf[sl], 0.0)
                return 0

            lax.fori_loop(0, C_B // 16, win, 0)
            return 0

        cp_p.wait()
        lax.cond(span <= BUF_B, narrow, wide, 0)
        pltpu.sync_copy(obuf, out_hbm.at[pl.ds(cb, C_B)])
        return 0

    lax.fori_loop(0, EPT // C_B, chunk, 0)


def _stage3(p_flat, segment_ids, thr, rden):
    f = pl.kernel(
        _apply_body,
        out_type=jax.ShapeDtypeStruct((N_EDGES,), jnp.float32),
        mesh=plsc.VectorSubcoreMesh(core_axis_name="c", subcore_axis_name="s"),
        scratch_types=[
            pltpu.VMEM((C_B,), jnp.float32),
            pltpu.VMEM((C_B,), jnp.int32),
            pltpu.VMEM((C_B,), jnp.float32),
            pltpu.VMEM((C_B,), jnp.float32),
            pltpu.VMEM((BUF_B,), jnp.float32),
            pltpu.VMEM((BUF_B,), jnp.float32),
            pltpu.VMEM((C_B,), jnp.float32),
            pltpu.SemaphoreType.DMA,
            pltpu.SemaphoreType.DMA,
        ],
        compiler_params=pltpu.CompilerParams(needs_layout_passes=False),
    )
    return f(p_flat, segment_ids, thr, rden)


def kernel(e_feats, segment_ids, v1, g1, b1, v2, g2, b2):
    w1 = g1[:, None] * v1 / jnp.linalg.norm(v1, axis=1, keepdims=True)
    w2 = (g2[:, None] * v2 / jnp.linalg.norm(v2, axis=1, keepdims=True))[0]
    w1t = w1.T
    w1b = jax.scipy.linalg.block_diag(*([w1t] * 8))
    w2b = jax.scipy.linalg.block_diag(*([w2[:, None]] * 8))
    b1t = jnp.tile(b1, 8)[None, :]
    b2t = jnp.full((1, 8), b2[0], jnp.float32)

    x = e_feats.reshape(ROWS, 128)
    p = _stage1(x, w1b, b1t, w2b, b2t).reshape(N_EDGES)
    ids = segment_ids.astype(jnp.int32)
    pair = _stage2(p, ids).reshape(SEG_PAD, 2)
    zpad = jnp.zeros((BUF_B,), jnp.float32)
    thr = jnp.concatenate([pair[:, 0], zpad])
    rden = jnp.concatenate([pair[:, 1], zpad])
    return _stage3(p, ids, thr, rden)


# speedup vs baseline: 1.0396x; 1.0396x over previous
import functools

import jax
import jax.numpy as jnp
from jax import lax
from jax.experimental import pallas as pl
from jax.experimental.pallas import tpu as pltpu
from jax.experimental.pallas import tpu_sc as plsc

N_NODES = 100000
N_EDGES = 3200000
EDGE_DIM = 16
HIDDEN = 8
TOPK = 16

NW = 32
SEG_PER_TILE = 3136
SEG_PAD = NW * SEG_PER_TILE

EDGE_BLK = 25600
ROWS_BLK = EDGE_BLK // 128

C_A = 8192
EPT = N_EDGES // NW
C_B = 4000
GW = 80

_I32MAX = 2147483647


def _mlp_body(feats_ref, aux_ref, p_ref):
    x = feats_ref[...]
    aux = aux_ref[...]
    w1t = aux[0:16, 0:8]
    b1 = aux[16, 0:8]
    w2 = aux[17, 0:8]
    b2 = aux[17, 8]
    h = jax.lax.dot(x, w1t, preferred_element_type=jnp.float32)
    h = jnp.maximum(h + b1[None, :], 0.0)
    s = jax.lax.dot(h, w2[:, None], preferred_element_type=jnp.float32)
    s = s[:, 0] + b2
    p_ref[...] = jnp.exp(s).reshape(ROWS_BLK, 128)


def _stage1(e_feats, aux):
    return pl.pallas_call(
        _mlp_body,
        grid=(N_EDGES // EDGE_BLK,),
        in_specs=[
            pl.BlockSpec((EDGE_BLK, EDGE_DIM), lambda i: (i, 0)),
            pl.BlockSpec((24, 128), lambda i: (0, 0)),
        ],
        out_specs=pl.BlockSpec((ROWS_BLK, 128), lambda i: (i, 0)),
        out_shape=jax.ShapeDtypeStruct((N_EDGES // 128, 128), jnp.float32),
    )(e_feats, aux)


def _seg_stats_body(p_hbm, ids_hbm, pair_hbm,
                    pbuf, ibuf, probe, ptab):
    wid = lax.axis_index("s") * 2 + lax.axis_index("c")
    s0 = wid * SEG_PER_TILE
    lane = lax.iota(jnp.int32, 16)
    zeros16 = jnp.zeros((16,), jnp.float32)

    def lower_bound(target):
        def body(_, st):
            l, h = st
            upd = h - l > 1
            mid = jnp.clip((l + h) // 2, 0, N_EDGES - 1)
            mid16 = pl.multiple_of((mid // 16) * 16, 16)
            pltpu.sync_copy(ids_hbm.at[pl.ds(mid16, 16)], probe)
            v16 = probe[...]
            v = jnp.sum(jnp.where(lane == mid - mid16, v16, 0))
            lt = v < target
            return (jnp.where(upd & lt, mid, l),
                    jnp.where(upd & (~lt), mid, h))

        _, h = lax.fori_loop(0, 24, body,
                             (jnp.int32(-1), jnp.int32(N_EDGES)))
        return h

    lo = lower_bound(s0)
    hi = lower_bound(s0 + SEG_PER_TILE)

    def zinit(i, _):
        ptab[pl.ds(i * 16, 16)] = zeros16
        return 0

    lax.fori_loop(0, 2 * SEG_PER_TILE // 16, zinit, 0)

    def close_seg(scur, sacc, top):
        idx = 2 * (scur - s0) + lane
        m01 = (lane < 2) & (scur >= s0)
        thr16 = jnp.full((16,), jnp.min(top), jnp.float32)
        rv = 1.0 / jnp.full((16,), jnp.sum(sacc), jnp.float32)
        vals = jnp.where(lane == 0, thr16, rv)
        plsc.store_scatter(ptab, [idx], vals, mask=m01)

    cbase0 = (lo // 16) * 16

    def chunk_step(k, st):
        cbase = pl.multiple_of(cbase0 + k * C_A, 16)

        def active(st):
            scur, sacc, top = st
            start = pl.multiple_of(
                jnp.minimum(cbase, jnp.int32(N_EDGES - C_A)), 16)
            off0 = cbase - start
            pltpu.sync_copy(p_hbm.at[pl.ds(start, C_A)],
                            pbuf.at[pl.ds(0, C_A)])
            pltpu.sync_copy(ids_hbm.at[pl.ds(start, C_A)],
                            ibuf.at[pl.ds(0, C_A)])

            def win_body(j, wst):
                scur, sacc, top = wst
                woff = off0 + j * 16
                ids_v = ibuf[pl.ds(woff, 16)]
                p_v = pbuf[pl.ds(woff, 16)]
                gidx = cbase + j * 16 + lane
                valid = (gidx >= lo) & (gidx < hi)

                def acc(scur, sacc, top):
                    pm = jnp.where(valid & (ids_v == scur), p_v, 0.0)
                    snew = lax.sort(pm)
                    top2 = lax.sort(jnp.maximum(top, lax.rev(snew, (0,))))
                    return sacc + pm, top2

                sacc, top = acc(scur, sacc, top)

                def seg_cond(st2):
                    return jnp.any(valid & (ids_v > st2[0]))

                def seg_body(st2):
                    scur, sacc, top = st2
                    close_seg(scur, sacc, top)
                    scur = jnp.min(jnp.where(valid & (ids_v > scur), ids_v,
                                             _I32MAX))
                    sacc, top = acc(scur, zeros16, zeros16)
                    return scur, sacc, top

                return lax.while_loop(seg_cond, seg_body, (scur, sacc, top))

            return lax.fori_loop(0, C_A // 16, win_body, (scur, sacc, top))

        return lax.cond(cbase < hi, active, lambda s: s, st)

    max_chunks = (N_EDGES + C_A - 1) // C_A + 1
    scur, sacc, top = lax.fori_loop(
        0, max_chunks, chunk_step,
        (jnp.int32(-1), zeros16, zeros16))
    close_seg(scur, sacc, top)

    s0a = pl.multiple_of(2 * s0, 64)
    pltpu.sync_copy(ptab, pair_hbm.at[pl.ds(s0a, 2 * SEG_PER_TILE)])


def _stage2(p_flat, segment_ids):
    f = pl.kernel(
        _seg_stats_body,
        out_type=jax.ShapeDtypeStruct((2 * SEG_PAD,), jnp.float32),
        mesh=plsc.VectorSubcoreMesh(core_axis_name="c", subcore_axis_name="s"),
        scratch_types=[
            pltpu.VMEM((2 * C_A,), jnp.float32),
            pltpu.VMEM((2 * C_A,), jnp.int32),
            pltpu.VMEM((16,), jnp.int32),
            pltpu.VMEM((2 * SEG_PER_TILE,), jnp.float32),
        ],
        compiler_params=pltpu.CompilerParams(needs_layout_passes=False),
    )
    return f(p_flat, segment_ids)


BUF_B = 8192


def _apply_body(p_hbm, ids_hbm, thr_hbm, rden_hbm, out_hbm,
                pbuf, ibuf, tbuf, rbuf, tsl, rsl, obuf, sem, sem2):
    wid = lax.axis_index("s") * 2 + lax.axis_index("c")
    base = wid * EPT
    lane = lax.iota(jnp.int32, 16)

    def chunk(kc, _):
        cb = pl.multiple_of(base + kc * C_B, 16)
        cp_p = pltpu.async_copy(p_hbm.at[pl.ds(cb, C_B)], pbuf, sem2)
        pltpu.sync_copy(ids_hbm.at[pl.ds(cb, C_B)], ibuf)
        first = jnp.sum(jnp.where(lane == 0, ibuf[pl.ds(0, 16)], 0))
        last = jnp.sum(jnp.where(lane == 15, ibuf[pl.ds(C_B - 16, 16)], 0))
        base8 = pl.multiple_of((first // 8) * 8, 8)
        span = last - base8 + 1

        def narrow(_):
            cp_t = pltpu.async_copy(thr_hbm.at[pl.ds(base8, BUF_B)], tsl, sem)
            pltpu.sync_copy(rden_hbm.at[pl.ds(base8, BUF_B)], rsl)
            cp_t.wait()

            def win(j, _):
                sl = pl.ds(j * 16, 16)
                pv = pbuf[sl]
                lidx = ibuf[sl] - base8
                tv = plsc.load_gather(tsl, [lidx])
                rv = plsc.load_gather(rsl, [lidx])
                obuf[sl] = jnp.where(pv >= tv, pv * rv, 0.0)
                return 0

            lax.fori_loop(0, C_B // 16, win, 0)
            return 0

        def wide(_):
            ngrp = C_B // GW
            for g0 in range(0, ngrp, 8):
                cps = []
                for g in range(g0, min(g0 + 8, ngrp)):
                    sl = pl.ds(g * GW, GW)
                    cps.append(pltpu.async_copy(thr_hbm.at[ibuf.at[sl]],
                                                tbuf.at[sl], sem))
                    cps.append(pltpu.async_copy(rden_hbm.at[ibuf.at[sl]],
                                                rbuf.at[sl], sem))
                for cp in cps:
                    cp.wait()

            def win(j, _):
                sl = pl.ds(j * 16, 16)
                pv = pbuf[sl]
                obuf[sl] = jnp.where(pv >= tbuf[sl], pv * rbuf[sl], 0.0)
                return 0

            lax.fori_loop(0, C_B // 16, win, 0)
            return 0

        cp_p.wait()
        lax.cond(span <= BUF_B, narrow, wide, 0)
        pltpu.sync_copy(obuf, out_hbm.at[pl.ds(cb, C_B)])
        return 0

    lax.fori_loop(0, EPT // C_B, chunk, 0)


def _stage3(p_flat, segment_ids, thr, rden):
    f = pl.kernel(
        _apply_body,
        out_type=jax.ShapeDtypeStruct((N_EDGES,), jnp.float32),
        mesh=plsc.VectorSubcoreMesh(core_axis_name="c", subcore_axis_name="s"),
        scratch_types=[
            pltpu.VMEM((C_B,), jnp.float32),
            pltpu.VMEM((C_B,), jnp.int32),
            pltpu.VMEM((C_B,), jnp.float32),
            pltpu.VMEM((C_B,), jnp.float32),
            pltpu.VMEM((BUF_B,), jnp.float32),
            pltpu.VMEM((BUF_B,), jnp.float32),
            pltpu.VMEM((C_B,), jnp.float32),
            pltpu.SemaphoreType.DMA,
            pltpu.SemaphoreType.DMA,
        ],
        compiler_params=pltpu.CompilerParams(needs_layout_passes=False),
    )
    return f(p_flat, segment_ids, thr, rden)


def kernel(e_feats, segment_ids, v1, g1, b1, v2, g2, b2):
    w1 = g1[:, None] * v1 / jnp.linalg.norm(v1, axis=1, keepdims=True)
    w2 = (g2[:, None] * v2 / jnp.linalg.norm(v2, axis=1, keepdims=True))[0]
    aux = jnp.zeros((24, 128), jnp.float32)
    aux = aux.at[0:16, 0:8].set(w1.T)
    aux = aux.at[16, 0:8].set(b1)
    aux = aux.at[17, 0:8].set(w2)
    aux = aux.at[17, 8].set(b2[0])

    p = _stage1(e_feats, aux).reshape(N_EDGES)
    ids = segment_ids.astype(jnp.int32)
    pair = _stage2(p, ids).reshape(SEG_PAD, 2)
    zpad = jnp.zeros((BUF_B,), jnp.float32)
    thr = jnp.concatenate([pair[:, 0], zpad])
    rden = jnp.concatenate([pair[:, 1], zpad])
    return _stage3(p, ids, thr, rden)
